# Initial kernel scaffold; baseline (speedup 1.0000x reference)
import jax
import jax.numpy as jnp
from jax.experimental import pallas as pl


def kernel(link_state, path_state, link_id, path_id, sequence, num_actions, W_ih_m, W_hh_m, b_ih_m, b_hh_m, W_ih_u, W_hh_u, b_ih_u, b_hh_u, W_r1, b_r1, W_r2, b_r2, W_out, b_out):
    raise NotImplementedError("write your pallas kernel here")


# speedup vs baseline: 2.5399x; 2.5399x over previous
import jax
import jax.numpy as jnp
from jax.experimental import pallas as pl

ML = 8
PAD = 48


def _gru(x, h, W_ih, W_hh, b_ih, b_hh):
    gi = x @ W_ih.T + b_ih
    gh = h @ W_hh.T + b_hh
    i_r, i_z, i_n = jnp.split(gi, 3, axis=-1)
    h_r, h_z, h_n = jnp.split(gh, 3, axis=-1)
    r = jax.nn.sigmoid(i_r + h_r)
    z = jax.nn.sigmoid(i_z + h_z)
    n = jnp.tanh(i_n + r * h_n)
    return (1.0 - z) * n + z * h


def kernel(link_state, path_state, link_id, path_id, sequence, num_actions,
           W_ih_m, W_hh_m, b_ih_m, b_hh_m, W_ih_u, W_hh_u, b_ih_u, b_hh_u,
           W_r1, b_r1, W_r2, b_r2, W_out, b_out):
    T = 4
    NL, F = link_state.shape
    NA = path_state.shape[0]
    E = link_id.shape[0]

    link_id = link_id.astype(jnp.int32)
    path_id = path_id.astype(jnp.int32)
    sequence = sequence.astype(jnp.int32)

    flat = sequence * NA + path_id
    eidx = jnp.arange(E, dtype=jnp.int32)
    slot_src = jnp.full((ML * NA,), -1, jnp.int32).at[flat].max(eidx)
    src_link = jnp.where(slot_src >= 0, link_id[jnp.maximum(slot_src, 0)], NL)
    cap = jnp.minimum(jnp.max(sequence) + 1, num_actions)

    zrows = jnp.zeros((PAD, F), jnp.float32)
    ls_ext = jnp.concatenate([link_state, zrows], axis=0)
    ps = path_state
    for _ in range(T):
        mi = ls_ext[src_link].reshape(ML, NA, F)
        h = ps
        outs = []
        for t in range(ML):
            h_new = _gru(mi[t], h, W_ih_m, W_hh_m, b_ih_m, b_hh_m)
            h = jnp.where(t < cap, h_new, h)
            outs.append(h)
        m_seq = jnp.concatenate(outs, axis=0)
        ps = outs[-1]
        gathered = m_seq[flat]
        m_agg = jnp.zeros((NL, F), jnp.float32).at[link_id].add(gathered)
        ls_new = _gru(m_agg, ls_ext[:NL], W_ih_u, W_hh_u, b_ih_u, b_hh_u)
        ls_ext = jnp.concatenate([ls_new, zrows], axis=0)
    h = jax.nn.selu(ps @ W_r1.T + b_r1)
    h = jax.nn.selu(h @ W_r2.T + b_r2)
    return h @ W_out.T + b_out


# speedup vs baseline: 2.8947x; 1.1397x over previous
import functools

import jax
import jax.numpy as jnp
from jax.experimental import pallas as pl
from jax.experimental.pallas import tpu as pltpu

ML = 8
F = 128
PAD = 48


def _gru_math(x, h, gi_w, gh_w, b_ih, b_hh):
    gi = jax.lax.dot_general(x, gi_w, (((1,), (0,)), ((), ())),
                             preferred_element_type=jnp.float32) + b_ih
    gh = jax.lax.dot_general(h, gh_w, (((1,), (0,)), ((), ())),
                             preferred_element_type=jnp.float32) + b_hh
    r = jax.nn.sigmoid(gi[:, :F] + gh[:, :F])
    z = jax.nn.sigmoid(gi[:, F:2 * F] + gh[:, F:2 * F])
    n = jnp.tanh(gi[:, 2 * F:] + r * gh[:, 2 * F:])
    return (1.0 - z) * n + z * h


def _scan_body(cap_ref, mi_ref, ps_ref, wih_ref, whh_ref, bih_ref, bhh_ref,
               out_ref, h_ref):
    t = pl.program_id(0)

    @pl.when(t == 0)
    def _():
        h_ref[...] = ps_ref[...]

    x = mi_ref[0]
    h = h_ref[...]
    h_new = _gru_math(x, h, wih_ref[...], whh_ref[...], bih_ref[...],
                      bhh_ref[...])
    keep = t < cap_ref[0]
    h2 = jnp.where(keep, h_new, h)
    h_ref[...] = h2
    out_ref[0] = h2


def _msg_scan(mi, ps, wih_t, whh_t, bih, bhh, cap):
    NA = ps.shape[0]
    grid = (ML,)
    return pl.pallas_call(
        _scan_body,
        grid_spec=pltpu.PrefetchScalarGridSpec(
            num_scalar_prefetch=1,
            grid=grid,
            in_specs=[
                pl.BlockSpec((1, NA, F), lambda t, *_: (t, 0, 0)),
                pl.BlockSpec((NA, F), lambda t, *_: (0, 0)),
                pl.BlockSpec((F, 3 * F), lambda t, *_: (0, 0)),
                pl.BlockSpec((F, 3 * F), lambda t, *_: (0, 0)),
                pl.BlockSpec((1, 3 * F), lambda t, *_: (0, 0)),
                pl.BlockSpec((1, 3 * F), lambda t, *_: (0, 0)),
            ],
            out_specs=pl.BlockSpec((1, NA, F), lambda t, *_: (t, 0, 0)),
            scratch_shapes=[pltpu.VMEM((NA, F), jnp.float32)],
        ),
        out_shape=jax.ShapeDtypeStruct((ML, NA, F), jnp.float32),
    )(cap, mi, ps, wih_t, whh_t, bih, bhh)


def _update_body(nl_ref, x_ref, h_ref, wih_ref, whh_ref, bih_ref, bhh_ref,
                 out_ref):
    i = pl.program_id(0)
    rows = x_ref.shape[0]
    x = x_ref[...]
    h = h_ref[...]
    h_new = _gru_math(x, h, wih_ref[...], whh_ref[...], bih_ref[...],
                      bhh_ref[...])
    row = jax.lax.broadcasted_iota(jnp.int32, (rows, 1), 0) + i * rows
    out_ref[...] = jnp.where(row < nl_ref[0], h_new, 0.0)


def _link_update(m_agg, ls_ext, wih_t, whh_t, bih, bhh, nl):
    NLp = ls_ext.shape[0]
    blk = NLp // 8
    return pl.pallas_call(
        _update_body,
        grid_spec=pltpu.PrefetchScalarGridSpec(
            num_scalar_prefetch=1,
            grid=(8,),
            in_specs=[
                pl.BlockSpec((blk, F), lambda i, *_: (i, 0)),
                pl.BlockSpec((blk, F), lambda i, *_: (i, 0)),
                pl.BlockSpec((F, 3 * F), lambda i, *_: (0, 0)),
                pl.BlockSpec((F, 3 * F), lambda i, *_: (0, 0)),
                pl.BlockSpec((1, 3 * F), lambda i, *_: (0, 0)),
                pl.BlockSpec((1, 3 * F), lambda i, *_: (0, 0)),
            ],
            out_specs=pl.BlockSpec((blk, F), lambda i, *_: (i, 0)),
            scratch_shapes=[],
        ),
        out_shape=jax.ShapeDtypeStruct((NLp, F), jnp.float32),
    )(nl, m_agg, ls_ext, wih_t, whh_t, bih, bhh)


def _selu(x):
    alpha = 1.6732632423543772848170429916717
    scale = 1.0507009873554804934193349852946
    return scale * jnp.where(x > 0, x, alpha * (jnp.exp(x) - 1.0))


def _readout_body(ps_ref, w1_ref, b1_ref, w2_ref, b2_ref, wo_ref, bo_ref,
                  out_ref):
    h = _selu(jax.lax.dot_general(ps_ref[...], w1_ref[...],
                                  (((1,), (0,)), ((), ())),
                                  preferred_element_type=jnp.float32)
              + b1_ref[...])
    h = _selu(jax.lax.dot_general(h, w2_ref[...], (((1,), (0,)), ((), ())),
                                  preferred_element_type=jnp.float32)
              + b2_ref[...])
    out_ref[...] = jax.lax.dot_general(h, wo_ref[...],
                                       (((1,), (0,)), ((), ())),
                                       preferred_element_type=jnp.float32) \
        + bo_ref[...]


def _readout(ps, w1_t, b1, w2_t, b2, wo_t, bo):
    NA = ps.shape[0]
    return pl.pallas_call(
        _readout_body,
        out_shape=jax.ShapeDtypeStruct((NA, F), jnp.float32),
    )(ps, w1_t, b1, w2_t, b2, wo_t, bo)


def _pad_body(ls_ref, out_ref):
    nl = ls_ref.shape[0]
    out_ref[:nl, :] = ls_ref[...]
    out_ref[nl:, :] = jnp.zeros_like(out_ref[nl:, :])


def _pad_links(ls, nlp):
    return pl.pallas_call(
        _pad_body,
        out_shape=jax.ShapeDtypeStruct((nlp, F), jnp.float32),
    )(ls)


def kernel(link_state, path_state, link_id, path_id, sequence, num_actions,
           W_ih_m, W_hh_m, b_ih_m, b_hh_m, W_ih_u, W_hh_u, b_ih_u, b_hh_u,
           W_r1, b_r1, W_r2, b_r2, W_out, b_out):
    T = 4
    NL = link_state.shape[0]
    NA = path_state.shape[0]
    E = link_id.shape[0]
    NLp = NL + PAD

    link_id = link_id.astype(jnp.int32)
    path_id = path_id.astype(jnp.int32)
    sequence = sequence.astype(jnp.int32)

    wm_ih = W_ih_m.T
    wm_hh = W_hh_m.T
    wu_ih = W_ih_u.T
    wu_hh = W_hh_u.T
    bm_ih = b_ih_m.reshape(1, -1)
    bm_hh = b_hh_m.reshape(1, -1)
    bu_ih = b_ih_u.reshape(1, -1)
    bu_hh = b_hh_u.reshape(1, -1)
    w1_t = W_r1.T
    w2_t = W_r2.T
    wo_t = jnp.pad(W_out.T, ((0, 0), (0, F - W_out.shape[0])))
    b1 = b_r1.reshape(1, -1)
    b2 = b_r2.reshape(1, -1)
    bo = jnp.pad(b_out.reshape(1, -1), ((0, 0), (0, F - b_out.shape[0])))

    flat = sequence * NA + path_id
    eidx = jnp.arange(E, dtype=jnp.int32)
    slot_src = jnp.full((ML * NA,), -1, jnp.int32).at[flat].max(eidx)
    src_link = jnp.where(slot_src >= 0, link_id[jnp.maximum(slot_src, 0)], NL)
    cap = jnp.minimum(jnp.max(sequence) + 1, num_actions).astype(jnp.int32)
    cap_arr = cap.reshape(1)
    nl_arr = jnp.full((1,), NL, jnp.int32)

    ls_ext = _pad_links(link_state, NLp)
    ps = path_state
    for _ in range(T):
        mi = ls_ext[src_link].reshape(ML, NA, F)
        m_seq = _msg_scan(mi, ps, wm_ih, wm_hh, bm_ih, bm_hh, cap_arr)
        ps = m_seq[ML - 1]
        gathered = m_seq.reshape(ML * NA, F)[flat]
        m_agg = jnp.zeros((NLp, F), jnp.float32).at[link_id].add(gathered)
        ls_ext = _link_update(m_agg, ls_ext, wu_ih, wu_hh, bu_ih, bu_hh,
                              nl_arr)
    out = _readout(ps, w1_t, b1, w2_t, b2, wo_t, bo)
    return out[:, :1]
